# baseline (speedup 1.0000x reference)
import functools

import jax
import jax.numpy as jnp
from jax import lax
from jax.experimental import pallas as pl
from jax.experimental.pallas import tpu as pltpu
from jax.experimental.pallas import tpu_sc as plsc

_NC = 2
_NS = 16
_NW = _NC * _NS
_CHUNK = 256
_NBUF = 2


def _gather_sc(idx_flat, table):
    n, = idx_flat.shape
    _, d = table.shape
    b_per_w = n // _NW
    n_chunks = b_per_w // _CHUNK
    n_outer = n_chunks // _NBUF
    mesh = plsc.VectorSubcoreMesh(core_axis_name="c", subcore_axis_name="s")

    @functools.partial(
        pl.kernel,
        mesh=mesh,
        out_type=jax.ShapeDtypeStruct((n, d), jnp.float32),
        scratch_types=(
            [
                pltpu.VMEM((b_per_w,), jnp.int32),
                pltpu.VMEM((_NBUF, _CHUNK, d), jnp.float32),
            ]
            + [pltpu.SemaphoreType.DMA] * (2 * _NBUF)
        ),
    )
    def k(idx_hbm, table_hbm, out_hbm, idx_v, rows_v, *sems):
        gsems = sems[:_NBUF]
        osems = sems[_NBUF:]
        wid = lax.axis_index("s") * _NC + lax.axis_index("c")
        base = wid * b_per_w
        pltpu.sync_copy(idx_hbm.at[pl.ds(base, b_per_w)], idx_v)

        def g_copy(j, b):
            return pltpu.make_async_copy(
                table_hbm.at[idx_v.at[pl.ds(j * _CHUNK, _CHUNK)]],
                rows_v.at[b],
                gsems[b],
            )

        def o_copy(j, b):
            return pltpu.make_async_copy(
                rows_v.at[b],
                out_hbm.at[pl.ds(base + j * _CHUNK, _CHUNK)],
                osems[b],
            )

        for b in range(_NBUF):
            g_copy(b, b).start()

        def outer(g, carry):
            j0 = g * _NBUF
            for b in range(_NBUF):
                j = j0 + b
                g_copy(j, b).wait()
                o_copy(j, b).start()
                o_copy(j, b).wait()
                g_copy(j + _NBUF, b).start()
            return carry

        lax.fori_loop(0, n_outer - 1, outer, 0)

        j0 = (n_outer - 1) * _NBUF
        for b in range(_NBUF):
            g_copy(j0 + b, b).wait()
            o_copy(j0 + b, b).start()
        for b in range(_NBUF):
            o_copy(j0 + b, b).wait()

    return k(idx_flat, table)


def kernel(actions, table):
    b, t = actions.shape
    flat = actions.reshape(b * t).astype(jnp.int32)
    out = _gather_sc(flat, table)
    return out.reshape(b, t, table.shape[1])


# speedup vs baseline: 1.0024x; 1.0024x over previous
import functools

import jax
import jax.numpy as jnp
from jax import lax
from jax.experimental import pallas as pl
from jax.experimental.pallas import tpu as pltpu
from jax.experimental.pallas import tpu_sc as plsc

_NC = 2
_NS = 16
_NW = _NC * _NS
_CHUNK = 128
_NBUF = 5


def _gather_sc(idx_flat, table):
    n, = idx_flat.shape
    _, d = table.shape
    b_per_w = n // _NW
    n_chunks = b_per_w // _CHUNK
    n_outer = n_chunks // _NBUF
    mesh = plsc.VectorSubcoreMesh(core_axis_name="c", subcore_axis_name="s")

    @functools.partial(
        pl.kernel,
        mesh=mesh,
        out_type=jax.ShapeDtypeStruct((n, d), jnp.float32),
        scratch_types=(
            [
                pltpu.VMEM((b_per_w,), jnp.int32),
                pltpu.VMEM((_NBUF, _CHUNK, d), jnp.float32),
            ]
            + [pltpu.SemaphoreType.DMA] * (2 * _NBUF)
        ),
    )
    def k(idx_hbm, table_hbm, out_hbm, idx_v, rows_v, *sems):
        gsems = sems[:_NBUF]
        osems = sems[_NBUF:]
        wid = lax.axis_index("s") * _NC + lax.axis_index("c")
        base = wid * b_per_w
        pltpu.sync_copy(idx_hbm.at[pl.ds(base, b_per_w)], idx_v)

        def g_copy(j, b):
            return pltpu.make_async_copy(
                table_hbm.at[idx_v.at[pl.ds(j * _CHUNK, _CHUNK)]],
                rows_v.at[b],
                gsems[b],
            )

        def o_copy(j, b):
            return pltpu.make_async_copy(
                rows_v.at[b],
                out_hbm.at[pl.ds(base + j * _CHUNK, _CHUNK)],
                osems[b],
            )

        for b in range(_NBUF):
            g_copy(b, b).start()

        def outer(g, carry):
            j0 = g * _NBUF
            for b in range(_NBUF):
                j = j0 + b
                g_copy(j, b).wait()
                o_copy(j, b).start()
                o_copy(j, b).wait()
                g_copy(j + _NBUF, b).start()
            return carry

        lax.fori_loop(0, n_outer - 1, outer, 0)

        j0 = (n_outer - 1) * _NBUF
        for b in range(_NBUF):
            g_copy(j0 + b, b).wait()
            o_copy(j0 + b, b).start()
        for b in range(_NBUF):
            o_copy(j0 + b, b).wait()

    return k(idx_flat, table)


def kernel(actions, table):
    b, t = actions.shape
    flat = actions.reshape(b * t).astype(jnp.int32)
    out = _gather_sc(flat, table)
    return out.reshape(b, t, table.shape[1])


# speedup vs baseline: 1.0372x; 1.0347x over previous
import functools

import jax
import jax.numpy as jnp
from jax import lax
from jax.experimental import pallas as pl
from jax.experimental.pallas import tpu as pltpu
from jax.experimental.pallas import tpu_sc as plsc

_NC = 2
_NS = 16
_NW = _NC * _NS
_CHUNK = 128
_NBUF = 4


def _gather_sc(idx_flat, table):
    n, = idx_flat.shape
    _, d = table.shape
    b_per_w = n // _NW
    n_chunks = b_per_w // _CHUNK
    n_outer = n_chunks // _NBUF
    mesh = plsc.VectorSubcoreMesh(core_axis_name="c", subcore_axis_name="s")

    @functools.partial(
        pl.kernel,
        mesh=mesh,
        out_type=jax.ShapeDtypeStruct((n, d), jnp.float32),
        scratch_types=(
            [
                pltpu.VMEM((b_per_w,), jnp.int32),
                pltpu.VMEM((_NBUF, _CHUNK, d), jnp.float32),
                pltpu.VMEM_SHARED((_NS, _NBUF // 2, _CHUNK, d), jnp.float32),
            ]
            + [pltpu.SemaphoreType.DMA] * (3 * _NBUF)
        ),
    )
    def k(idx_hbm, table_hbm, out_hbm, idx_v, rows_v, sp_v, *sems):
        gsems = sems[:_NBUF]
        osems = sems[_NBUF:2 * _NBUF]
        csems = sems[2 * _NBUF:]
        cid = lax.axis_index("c")
        sid = lax.axis_index("s")
        wid = sid * _NC + cid
        base = wid * b_per_w
        pltpu.sync_copy(idx_hbm.at[pl.ds(base, b_per_w)], idx_v)

        def g_copy(j, b):
            return pltpu.make_async_copy(
                table_hbm.at[idx_v.at[pl.ds(j * _CHUNK, _CHUNK)]],
                rows_v.at[b],
                gsems[b],
            )

        def x_copy(b):
            return pltpu.make_async_copy(
                rows_v.at[b],
                sp_v.at[sid, b // 2],
                csems[b],
            )

        def o_copy(j, b):
            src = sp_v.at[sid, b // 2] if b % 2 else rows_v.at[b]
            return pltpu.make_async_copy(
                src,
                out_hbm.at[pl.ds(base + j * _CHUNK, _CHUNK)],
                osems[b],
            )

        def emit_slot(j, b, refill):
            g_copy(j, b).wait()
            if b % 2:
                x_copy(b).start()
                x_copy(b).wait()
                o_copy(j, b).start()
                if refill:
                    g_copy(j + _NBUF, b).start()
                o_copy(j, b).wait()
            else:
                o_copy(j, b).start()
                o_copy(j, b).wait()
                if refill:
                    g_copy(j + _NBUF, b).start()

        for b in range(_NBUF):
            g_copy(b, b).start()

        def outer(g, carry):
            j0 = g * _NBUF
            for b in range(_NBUF):
                emit_slot(j0 + b, b, True)
            return carry

        lax.fori_loop(0, n_outer - 1, outer, 0)

        j0 = (n_outer - 1) * _NBUF
        for b in range(_NBUF):
            emit_slot(j0 + b, b, False)

    return k(idx_flat, table)


def kernel(actions, table):
    b, t = actions.shape
    flat = actions.reshape(b * t).astype(jnp.int32)
    out = _gather_sc(flat, table)
    return out.reshape(b, t, table.shape[1])


# speedup vs baseline: 1.0395x; 1.0022x over previous
import functools

import jax
import jax.numpy as jnp
from jax import lax
from jax.experimental import pallas as pl
from jax.experimental.pallas import tpu as pltpu
from jax.experimental.pallas import tpu_sc as plsc

_NC = 2
_NS = 16
_NW = _NC * _NS
_CHUNK = 128
_NBUF = 4
_SP_SLOTS = (1, 3)


def _gather_sc(idx_flat, table):
    n, = idx_flat.shape
    _, d = table.shape
    b_per_w = n // _NW
    n_chunks = b_per_w // _CHUNK
    n_outer = n_chunks // _NBUF
    mesh = plsc.VectorSubcoreMesh(core_axis_name="c", subcore_axis_name="s")

    @functools.partial(
        pl.kernel,
        mesh=mesh,
        out_type=jax.ShapeDtypeStruct((n, d), jnp.float32),
        scratch_types=(
            [
                pltpu.VMEM((b_per_w,), jnp.int32),
                pltpu.VMEM((_NBUF, _CHUNK, d), jnp.float32),
                pltpu.VMEM_SHARED((_NS, len(_SP_SLOTS), _CHUNK, d), jnp.float32),
            ]
            + [pltpu.SemaphoreType.DMA] * (3 * _NBUF)
        ),
    )
    def k(idx_hbm, table_hbm, out_hbm, idx_v, rows_v, sp_v, *sems):
        gsems = sems[:_NBUF]
        osems = sems[_NBUF:2 * _NBUF]
        csems = sems[2 * _NBUF:]
        cid = lax.axis_index("c")
        sid = lax.axis_index("s")
        wid = sid * _NC + cid
        base = wid * b_per_w
        pltpu.sync_copy(idx_hbm.at[pl.ds(base, b_per_w)], idx_v)

        def g_copy(j, b):
            return pltpu.make_async_copy(
                table_hbm.at[idx_v.at[pl.ds(j * _CHUNK, _CHUNK)]],
                rows_v.at[b],
                gsems[b],
            )

        def x_copy(b):
            return pltpu.make_async_copy(
                rows_v.at[b],
                sp_v.at[sid, _SP_SLOTS.index(b)],
                csems[b],
            )

        def o_copy(j, b):
            if b in _SP_SLOTS:
                src = sp_v.at[sid, _SP_SLOTS.index(b)]
            else:
                src = rows_v.at[b]
            return pltpu.make_async_copy(
                src,
                out_hbm.at[pl.ds(base + j * _CHUNK, _CHUNK)],
                osems[b],
            )

        def emit_slot(j, b, refill):
            g_copy(j, b).wait()
            if b in _SP_SLOTS:
                x_copy(b).start()
                x_copy(b).wait()
                o_copy(j, b).start()
                if refill:
                    g_copy(j + _NBUF, b).start()
                o_copy(j, b).wait()
            else:
                o_copy(j, b).start()
                o_copy(j, b).wait()
                if refill:
                    g_copy(j + _NBUF, b).start()

        for b in range(_NBUF):
            g_copy(b, b).start()

        def outer(g, carry):
            j0 = g * _NBUF
            for b in range(_NBUF):
                emit_slot(j0 + b, b, True)
            return carry

        lax.fori_loop(0, n_outer - 1, outer, 0)

        j0 = (n_outer - 1) * _NBUF
        for b in range(_NBUF):
            emit_slot(j0 + b, b, False)

    return k(idx_flat, table)


def kernel(actions, table):
    b, t = actions.shape
    flat = actions.reshape(b * t).astype(jnp.int32)
    out = _gather_sc(flat, table)
    return out.reshape(b, t, table.shape[1])
